# Initial kernel scaffold; baseline (speedup 1.0000x reference)
import jax
import jax.numpy as jnp
from jax.experimental import pallas as pl


def kernel(alleles, positions, table):
    raise NotImplementedError("write your pallas kernel here")


# speedup vs baseline: 3.1040x; 3.1040x over previous
import functools

import jax
import jax.numpy as jnp
from jax import lax
from jax.experimental import pallas as pl
from jax.experimental.pallas import tpu as pltpu
from jax.experimental.pallas import tpu_sc as plsc

_NALLELES = 10
_D = 32
_L = 16
_NC = 2
_NS = 16
_NW = _NC * _NS


def _sc_embed(pos_flat, a0_flat, a1_flat, table, n_rows, chunk):
  per_w = n_rows // _NW
  n_chunks = per_w // chunk
  mesh = plsc.VectorSubcoreMesh(core_axis_name="c", subcore_axis_name="s")

  @functools.partial(
      pl.kernel,
      mesh=mesh,
      out_type=jax.ShapeDtypeStruct((n_rows, _D), jnp.float32),
      compiler_params=pltpu.CompilerParams(use_tc_tiling_on_sc=False),
      scratch_types=[
          pltpu.VMEM((chunk,), jnp.int32),
          pltpu.VMEM((chunk,), jnp.int32),
          pltpu.VMEM((chunk,), jnp.int32),
          pltpu.VMEM((chunk,), jnp.int32),
          pltpu.VMEM((chunk,), jnp.int32),
          pltpu.VMEM((chunk, _D), jnp.float32),
          pltpu.VMEM((chunk, _D), jnp.float32),
          pltpu.SemaphoreType.DMA,
          pltpu.SemaphoreType.DMA,
      ],
  )
  def k(pos_hbm, a0_hbm, a1_hbm, table_hbm, out_hbm,
        pos_v, a0_v, a1_v, idx0_v, idx1_v, buf0, buf1, sem0, sem1):
    wid = lax.axis_index("s") * _NC + lax.axis_index("c")
    w_base = wid * per_w

    def chunk_body(t, _):
      base = w_base + t * chunk
      pltpu.sync_copy(pos_hbm.at[pl.ds(base, chunk)], pos_v)
      pltpu.sync_copy(a0_hbm.at[pl.ds(base, chunk)], a0_v)
      pltpu.sync_copy(a1_hbm.at[pl.ds(base, chunk)], a1_v)

      def idx_body(j, _):
        s = j * _L
        p = pos_v[pl.ds(s, _L)] * _NALLELES
        idx0_v[pl.ds(s, _L)] = p + a0_v[pl.ds(s, _L)]
        idx1_v[pl.ds(s, _L)] = p + a1_v[pl.ds(s, _L)]
        return 0

      lax.fori_loop(0, chunk // _L, idx_body, 0, unroll=4)

      cp0 = pltpu.async_copy(table_hbm.at[idx0_v], buf0, sem0)
      cp1 = pltpu.async_copy(table_hbm.at[idx1_v], buf1, sem1)
      cp0.wait()
      cp1.wait()

      def add_body(i, _):
        buf0[i, pl.ds(0, _L)] = buf0[i, pl.ds(0, _L)] + buf1[i, pl.ds(0, _L)]
        buf0[i, pl.ds(_L, _L)] = (
            buf0[i, pl.ds(_L, _L)] + buf1[i, pl.ds(_L, _L)])
        return 0

      lax.fori_loop(0, chunk, add_body, 0, unroll=4)

      pltpu.sync_copy(buf0, out_hbm.at[pl.ds(base, chunk)])
      return 0

    lax.fori_loop(0, n_chunks, chunk_body, 0)

  return k(pos_flat, a0_flat, a1_flat, table)


def kernel(alleles, positions, table):
  b, s, _ = alleles.shape
  n = b * s
  pos_flat = positions.reshape(n)
  al = alleles.reshape(n, 2)
  a0 = al[:, 0]
  a1 = al[:, 1]
  out = _sc_embed(pos_flat, a0, a1, table, n, 512)
  return out.reshape(b, s, _D)


# speedup vs baseline: 3.5754x; 1.1519x over previous
import functools

import jax
import jax.numpy as jnp
from jax import lax
from jax.experimental import pallas as pl
from jax.experimental.pallas import tpu as pltpu
from jax.experimental.pallas import tpu_sc as plsc

_NALLELES = 10
_D = 32
_L = 16
_NC = 2
_NS = 16
_NW = _NC * _NS


def _sc_embed(pos_flat, a0_flat, a1_flat, table, n_rows, chunk):
  per_w = n_rows // _NW
  n_chunks = per_w // chunk
  assert per_w % chunk == 0 and n_chunks % 2 == 0 and n_chunks >= 6
  mesh = plsc.VectorSubcoreMesh(core_axis_name="c", subcore_axis_name="s")

  idx_t = pltpu.VMEM((chunk,), jnp.int32)
  row_t = pltpu.VMEM((chunk, _D), jnp.float32)

  @functools.partial(
      pl.kernel,
      mesh=mesh,
      out_type=jax.ShapeDtypeStruct((n_rows, _D), jnp.float32),
      compiler_params=pltpu.CompilerParams(use_tc_tiling_on_sc=False),
      scratch_types=[idx_t] * 10 + [row_t] * 4 + [pltpu.SemaphoreType.DMA] * 6,
  )
  def k(pos_hbm, a0_hbm, a1_hbm, table_hbm, out_hbm,
        pos0, pos1, al00, al01, al10, al11,
        ix00, ix01, ix10, ix11, r00, r01, r10, r11,
        isem0, isem1, gsem0, gsem1, wsem0, wsem1):
    pos_v = (pos0, pos1)
    a0_v = (al00, al01)
    a1_v = (al10, al11)
    ix0_v = (ix00, ix01)
    ix1_v = (ix10, ix11)
    r0_v = (r00, r01)
    r1_v = (r10, r11)
    isem = (isem0, isem1)
    gsem = (gsem0, gsem1)
    wsem = (wsem0, wsem1)

    wid = lax.axis_index("s") * _NC + lax.axis_index("c")
    w_base = wid * per_w

    def issue_in(t, b):
      base = w_base + t * chunk
      pltpu.async_copy(pos_hbm.at[pl.ds(base, chunk)], pos_v[b], isem[b])
      pltpu.async_copy(a0_hbm.at[pl.ds(base, chunk)], a0_v[b], isem[b])
      pltpu.async_copy(a1_hbm.at[pl.ds(base, chunk)], a1_v[b], isem[b])

    def wait_in(b):
      pltpu.make_async_copy(pos_hbm.at[pl.ds(0, chunk)], pos_v[b],
                            isem[b]).wait()
      pltpu.make_async_copy(a0_hbm.at[pl.ds(0, chunk)], a0_v[b],
                            isem[b]).wait()
      pltpu.make_async_copy(a1_hbm.at[pl.ds(0, chunk)], a1_v[b],
                            isem[b]).wait()

    def compute_idx(b):
      def body(j, _):
        s = j * _L
        p = pos_v[b][pl.ds(s, _L)] * _NALLELES
        ix0_v[b][pl.ds(s, _L)] = p + a0_v[b][pl.ds(s, _L)]
        ix1_v[b][pl.ds(s, _L)] = p + a1_v[b][pl.ds(s, _L)]
        return 0

      lax.fori_loop(0, chunk // _L, body, 0, unroll=4)

    def issue_gather(b):
      pltpu.async_copy(table_hbm.at[ix0_v[b]], r0_v[b], gsem[b])
      pltpu.async_copy(table_hbm.at[ix1_v[b]], r1_v[b], gsem[b])

    def wait_gather(b):
      pltpu.make_async_copy(out_hbm.at[pl.ds(0, chunk)], r0_v[b],
                            gsem[b]).wait()
      pltpu.make_async_copy(out_hbm.at[pl.ds(0, chunk)], r1_v[b],
                            gsem[b]).wait()

    def add_rows(b):
      def body(i, _):
        r0_v[b][i, pl.ds(0, _L)] = (
            r0_v[b][i, pl.ds(0, _L)] + r1_v[b][i, pl.ds(0, _L)])
        r0_v[b][i, pl.ds(_L, _L)] = (
            r0_v[b][i, pl.ds(_L, _L)] + r1_v[b][i, pl.ds(_L, _L)])
        return 0

      lax.fori_loop(0, chunk, body, 0, unroll=4)

    def issue_wb(t, b):
      base = w_base + t * chunk
      pltpu.async_copy(r0_v[b], out_hbm.at[pl.ds(base, chunk)], wsem[b])

    def wait_wb(b):
      pltpu.make_async_copy(r0_v[b], out_hbm.at[pl.ds(0, chunk)],
                            wsem[b]).wait()

    issue_in(0, 0)
    issue_in(1, 1)
    wait_in(0)
    compute_idx(0)
    issue_in(2, 0)
    issue_gather(0)
    wait_in(1)
    compute_idx(1)
    issue_in(3, 1)
    issue_gather(1)
    wait_gather(0)
    add_rows(0)
    issue_wb(0, 0)

    def pair_body(g, _):
      for b in range(2):
        t = 2 * g + b
        wait_in(b)
        compute_idx(b)
        issue_in(t + 2, b)
        wait_wb(b)
        issue_gather(b)
        wait_gather(1 - b)
        add_rows(1 - b)
        issue_wb(t - 1, 1 - b)
      return 0

    lax.fori_loop(1, n_chunks // 2 - 1, pair_body, 0)

    tl = n_chunks - 2
    wait_in(0)
    compute_idx(0)
    wait_wb(0)
    issue_gather(0)
    wait_gather(1)
    add_rows(1)
    issue_wb(tl - 1, 1)

    wait_in(1)
    compute_idx(1)
    wait_wb(1)
    issue_gather(1)
    wait_gather(0)
    add_rows(0)
    issue_wb(tl, 0)

    wait_gather(1)
    add_rows(1)
    issue_wb(tl + 1, 1)
    wait_wb(0)
    wait_wb(1)

  return k(pos_flat, a0_flat, a1_flat, table)


def kernel(alleles, positions, table):
  b, s, _ = alleles.shape
  n = b * s
  pos_flat = positions.reshape(n)
  al = alleles.reshape(n, 2)
  a0 = al[:, 0]
  a1 = al[:, 1]
  out = _sc_embed(pos_flat, a0, a1, table, n, 800)
  return out.reshape(b, s, _D)
